# baseline (speedup 1.0000x reference)
import functools

import jax
import jax.numpy as jnp
from jax import lax
from jax.experimental import pallas as pl
from jax.experimental.pallas import tpu as pltpu
from jax.experimental.pallas import tpu_sc as plsc

B = 16384
L = 50
D = 128
NC = 2
NS = 16
NW = NC * NS
E_PER_W = B // NW
CHUNK_E = 16
ROWS = CHUNK_E * L
N_CHUNKS = E_PER_W // CHUNK_E
IDXC = 2

VOCAB = 40961
PACK_CHUNK = 64
PACK_PER_W = -(-VOCAB // NW)
PACK_STEPS = -(-PACK_PER_W // PACK_CHUNK)


def _pack_table_sc(table_flat):
    mesh = plsc.VectorSubcoreMesh(core_axis_name="c", subcore_axis_name="s")

    @functools.partial(
        pl.kernel,
        out_type=jax.ShapeDtypeStruct((VOCAB, D // 2), jnp.int32),
        mesh=mesh,
        compiler_params=pltpu.CompilerParams(
            needs_layout_passes=False, use_tc_tiling_on_sc=False),
        scratch_types=[
            pltpu.VMEM((PACK_CHUNK * D,), jnp.float32),
            pltpu.VMEM((PACK_CHUNK * D,), jnp.float32),
            pltpu.VMEM((PACK_CHUNK, D // 2), jnp.int32),
            pltpu.VMEM((PACK_CHUNK, D // 2), jnp.int32),
            pltpu.SemaphoreType.DMA,
            pltpu.SemaphoreType.DMA,
            pltpu.SemaphoreType.DMA,
            pltpu.SemaphoreType.DMA,
        ],
    )
    def k(tflat_hbm, out_hbm, fin0, fin1, pout0, pout1,
          semA, semB, semOA, semOB):
        wid = lax.axis_index("s") * NC + lax.axis_index("c")
        fins, sems = [fin0, fin1], [semA, semB]
        pouts, osems = [pout0, pout1], [semOA, semOB]

        def src_row(t):
            return jnp.minimum(wid * PACK_PER_W + t * PACK_CHUNK,
                               VOCAB - PACK_CHUNK)

        def fire(t, buf):
            pltpu.async_copy(
                tflat_hbm.at[pl.ds(src_row(t) * D, PACK_CHUNK * D)],
                fins[buf], sems[buf])

        def drain(buf):
            pltpu.make_async_copy(
                tflat_hbm.at[pl.ds(0, PACK_CHUNK * D)],
                fins[buf], sems[buf]).wait()

        def drain_out(buf):
            pltpu.make_async_copy(
                pouts[buf], out_hbm.at[pl.ds(0, PACK_CHUNK)],
                osems[buf]).wait()

        def pack_step(fin_v, pout_v):
            for r in range(PACK_CHUNK):
                for g in range(D // 32):
                    a = fin_v[pl.ds(r * D + g * 32, 16)]
                    b = fin_v[pl.ds(r * D + g * 32 + 16, 16)]
                    packed = plsc.pack(
                        a, b, format=plsc.PackFormat.INTERLEAVED)
                    pout_v[r, pl.ds(g * 16, 16)] = plsc.bitcast(
                        packed, jnp.int32)

        fire(0, 0)

        def step2(u, carry):
            for par in range(2):
                t = 2 * u + par
                fire(jnp.minimum(t + 1, PACK_STEPS - 1), 1 - par)
                drain(par)

                @pl.when(u > 0)
                def _():
                    drain_out(par)

                pack_step(fins[par], pouts[par])
                pltpu.async_copy(pouts[par],
                                 out_hbm.at[pl.ds(src_row(t), PACK_CHUNK)],
                                 osems[par])
            return carry

        lax.fori_loop(0, PACK_STEPS // 2, step2, 0)
        t = PACK_STEPS - 1
        drain(t % 2)
        drain_out(0)
        pack_step(fins[t % 2], pouts[0])
        pltpu.sync_copy(pouts[0], out_hbm.at[pl.ds(src_row(t), PACK_CHUNK)])
        drain_out(1)

    return k(table_flat)


def _emb_pool_sc(xw_flat, xb_flat, table_i32, elem_lo, nb):
    e_per_w = nb // NW
    n_chunks = e_per_w // CHUNK_E
    mesh = plsc.VectorSubcoreMesh(core_axis_name="c", subcore_axis_name="s")

    @functools.partial(
        pl.kernel,
        out_type=jax.ShapeDtypeStruct((nb // CHUNK_E, 2, CHUNK_E, D),
                                      jnp.float32),
        mesh=mesh,
        compiler_params=pltpu.CompilerParams(
            needs_layout_passes=False, use_tc_tiling_on_sc=False),
        scratch_types=[
            pltpu.VMEM((IDXC * ROWS,), jnp.int32),
            pltpu.VMEM((IDXC * ROWS,), jnp.int32),
            pltpu.VMEM((ROWS, D // 2), jnp.int32),
            pltpu.VMEM((ROWS, D // 2), jnp.int32),
            pltpu.VMEM((2, CHUNK_E, D), jnp.float32),
            pltpu.SemaphoreType.DMA,
            pltpu.SemaphoreType.DMA,
        ],
    )
    def k(xw_hbm, xb_hbm, table_hbm, out_hbm,
          idx0, idx1, rows0, rows1, outc_v, sem0, sem1):
        wid = lax.axis_index("s") * NC + lax.axis_index("c")
        w_base = elem_lo + wid * e_per_w
        idx_b, rows_b, sems = [idx0, idx1], [rows0, rows1], [sem0, sem1]
        srcs = [xw_hbm, xb_hbm]

        def fire(buf, chunk):
            @pl.when(chunk % IDXC == 0)
            def _():
                idx_base = (w_base + chunk * CHUNK_E) * L
                pltpu.sync_copy(
                    srcs[buf].at[pl.ds(idx_base, IDXC * ROWS)], idx_b[buf])

            slot = (chunk % IDXC) * ROWS
            off = 0
            while off < ROWS:
                n = min(128, ROWS - off)
                pltpu.async_copy(
                    table_hbm.at[idx_b[buf].at[pl.ds(slot + off, n)]],
                    rows_b[buf].at[pl.ds(off, n)], sems[buf])
                off += n

        def drain(buf):
            pltpu.make_async_copy(
                table_hbm.at[pl.ds(0, ROWS)], rows_b[buf], sems[buf]).wait()

        def unpack_row(rows_v, r):
            acc = []
            for g in range(D // 32):
                packed = plsc.bitcast(
                    rows_v[r, pl.ds(g * 16, 16)], jnp.bfloat16)
                lo, hi = plsc.unpack(
                    packed, format=plsc.PackFormat.INTERLEAVED)
                acc += [lo, hi]
            return tuple(acc)

        def reduce_side(buf):
            rows_v = rows_b[buf]
            for e in range(CHUNK_E):
                r0 = e * L

                def body7(t, acc, r0=r0, rows_v=rows_v):
                    j = 1 + t * 7
                    for u in range(7):
                        vals = unpack_row(rows_v, r0 + j + u)
                        acc = tuple(a + v for a, v in zip(acc, vals))
                    return acc

                acc = unpack_row(rows_v, r0)
                acc = lax.fori_loop(0, (L - 1) // 7, body7, acc)
                for d in range(D // 16):
                    outc_v[buf, e, pl.ds(d * 16, 16)] = acc[d]

        fire(0, 0)

        def chunk_body(c, carry):
            fire(1, c)
            drain(0)
            reduce_side(0)
            fire(0, jnp.minimum(c + 1, n_chunks - 1))
            drain(1)
            reduce_side(1)
            pltpu.sync_copy(outc_v, out_hbm.at[wid * n_chunks + c])
            return carry

        lax.fori_loop(0, n_chunks, chunk_body, 0)
        drain(0)

    return k(xw_flat, xb_flat, table_i32)


def _mlp_tc(x4, W2, b2, W3, b3, W4, b4):
    BLK = 2048
    BC = BLK // CHUNK_E

    def body(x_ref, w2_ref, b2_ref, w3_ref, b3_ref, w4_ref, b4_ref, o_ref):
        xw = jnp.maximum(x_ref[:, 0].reshape(BLK, D), 0.0)
        xb = jnp.maximum(x_ref[:, 1].reshape(BLK, D), 0.0)
        h = (jnp.dot(xw, w2_ref[:D], preferred_element_type=jnp.float32)
             + jnp.dot(xb, w2_ref[D:], preferred_element_type=jnp.float32))
        h = jnp.maximum(h + b2_ref[...], 0.0)
        h = jnp.dot(h, w3_ref[...], preferred_element_type=jnp.float32)
        h = jnp.maximum(h + b3_ref[...], 0.0)
        h = jnp.dot(h, w4_ref[...], preferred_element_type=jnp.float32)
        o_ref[...] = h + b4_ref[...]

    nb = x4.shape[0] * CHUNK_E
    return pl.pallas_call(
        body,
        grid=(nb // BLK,),
        in_specs=[
            pl.BlockSpec((BC, 2, CHUNK_E, D), lambda i: (i, 0, 0, 0)),
            pl.BlockSpec((2 * D, 32), lambda i: (0, 0)),
            pl.BlockSpec((1, 32), lambda i: (0, 0)),
            pl.BlockSpec((32, 32), lambda i: (0, 0)),
            pl.BlockSpec((1, 32), lambda i: (0, 0)),
            pl.BlockSpec((32, 1), lambda i: (0, 0)),
            pl.BlockSpec((1, 1), lambda i: (0, 0)),
        ],
        out_specs=pl.BlockSpec((BLK, 1), lambda i: (i, 0)),
        out_shape=jax.ShapeDtypeStruct((nb, 1), jnp.float32),
    )(x4, W2, b2.reshape(1, 32), W3, b3.reshape(1, 32), W4, b4.reshape(1, 1))


def kernel(x_w, x_b, table, W2, b2, W3, b3, W4, b4):
    xw_flat = x_w.astype(jnp.int32).reshape(-1)
    xb_flat = x_b.astype(jnp.int32).reshape(-1)
    table_i32 = _pack_table_sc(table.reshape(-1))
    halves = []
    for h in range(2):
        pooled = _emb_pool_sc(xw_flat, xb_flat, table_i32,
                              h * (B // 2), B // 2)
        halves.append(_mlp_tc(pooled, W2, b2, W3, b3, W4, b4))
    return jnp.concatenate(halves, axis=0)


# speedup vs baseline: 1.0257x; 1.0257x over previous
import functools

import jax
import jax.numpy as jnp
from jax import lax
from jax.experimental import pallas as pl
from jax.experimental.pallas import tpu as pltpu
from jax.experimental.pallas import tpu_sc as plsc

B = 16384
L = 50
D = 128
NC = 2
NS = 16
NW = NC * NS
E_PER_W = B // NW
CHUNK_E = 8
ROWS = CHUNK_E * L
N_CHUNKS = E_PER_W // CHUNK_E
IDXC = 4

VOCAB = 40961
PACK_CHUNK = 64
PACK_PER_W = -(-VOCAB // NW)
PACK_STEPS = -(-PACK_PER_W // PACK_CHUNK)


def _pack_table_sc(table_flat):
    mesh = plsc.VectorSubcoreMesh(core_axis_name="c", subcore_axis_name="s")

    @functools.partial(
        pl.kernel,
        out_type=jax.ShapeDtypeStruct((VOCAB, D // 2), jnp.int32),
        mesh=mesh,
        compiler_params=pltpu.CompilerParams(
            needs_layout_passes=False, use_tc_tiling_on_sc=False),
        scratch_types=[
            pltpu.VMEM((PACK_CHUNK * D,), jnp.float32),
            pltpu.VMEM((PACK_CHUNK * D,), jnp.float32),
            pltpu.VMEM((PACK_CHUNK, D // 2), jnp.int32),
            pltpu.VMEM((PACK_CHUNK, D // 2), jnp.int32),
            pltpu.SemaphoreType.DMA,
            pltpu.SemaphoreType.DMA,
            pltpu.SemaphoreType.DMA,
            pltpu.SemaphoreType.DMA,
        ],
    )
    def k(tflat_hbm, out_hbm, fin0, fin1, pout0, pout1,
          semA, semB, semOA, semOB):
        wid = lax.axis_index("s") * NC + lax.axis_index("c")
        fins, sems = [fin0, fin1], [semA, semB]
        pouts, osems = [pout0, pout1], [semOA, semOB]

        def src_row(t):
            return jnp.minimum(wid * PACK_PER_W + t * PACK_CHUNK,
                               VOCAB - PACK_CHUNK)

        def fire(t, buf):
            pltpu.async_copy(
                tflat_hbm.at[pl.ds(src_row(t) * D, PACK_CHUNK * D)],
                fins[buf], sems[buf])

        def drain(buf):
            pltpu.make_async_copy(
                tflat_hbm.at[pl.ds(0, PACK_CHUNK * D)],
                fins[buf], sems[buf]).wait()

        def drain_out(buf):
            pltpu.make_async_copy(
                pouts[buf], out_hbm.at[pl.ds(0, PACK_CHUNK)],
                osems[buf]).wait()

        def pack_step(fin_v, pout_v):
            for r in range(PACK_CHUNK):
                for g in range(D // 32):
                    a = fin_v[pl.ds(r * D + g * 32, 16)]
                    b = fin_v[pl.ds(r * D + g * 32 + 16, 16)]
                    packed = plsc.pack(
                        a, b, format=plsc.PackFormat.INTERLEAVED)
                    pout_v[r, pl.ds(g * 16, 16)] = plsc.bitcast(
                        packed, jnp.int32)

        fire(0, 0)

        def step2(u, carry):
            for par in range(2):
                t = 2 * u + par
                fire(jnp.minimum(t + 1, PACK_STEPS - 1), 1 - par)
                drain(par)

                @pl.when(u > 0)
                def _():
                    drain_out(par)

                pack_step(fins[par], pouts[par])
                pltpu.async_copy(pouts[par],
                                 out_hbm.at[pl.ds(src_row(t), PACK_CHUNK)],
                                 osems[par])
            return carry

        lax.fori_loop(0, PACK_STEPS // 2, step2, 0)
        t = PACK_STEPS - 1
        drain(t % 2)
        drain_out(0)
        pack_step(fins[t % 2], pouts[0])
        pltpu.sync_copy(pouts[0], out_hbm.at[pl.ds(src_row(t), PACK_CHUNK)])
        drain_out(1)

    return k(table_flat)


def _emb_pool_sc(xw_flat, xb_flat, table_i32, elem_lo, nb):
    e_per_w = nb // NW
    n_chunks = e_per_w // CHUNK_E
    mesh = plsc.VectorSubcoreMesh(core_axis_name="c", subcore_axis_name="s")

    @functools.partial(
        pl.kernel,
        out_type=jax.ShapeDtypeStruct((nb // CHUNK_E, 2, CHUNK_E, D),
                                      jnp.float32),
        mesh=mesh,
        compiler_params=pltpu.CompilerParams(
            needs_layout_passes=False, use_tc_tiling_on_sc=False),
        scratch_types=[
            pltpu.VMEM((IDXC * ROWS,), jnp.int32),
            pltpu.VMEM((IDXC * ROWS,), jnp.int32),
            pltpu.VMEM((ROWS, D // 2), jnp.int32),
            pltpu.VMEM((ROWS, D // 2), jnp.int32),
            pltpu.VMEM((2, CHUNK_E, D), jnp.float32),
            pltpu.SemaphoreType.DMA,
            pltpu.SemaphoreType.DMA,
        ],
    )
    def k(xw_hbm, xb_hbm, table_hbm, out_hbm,
          idx0, idx1, rows0, rows1, outc_v, sem0, sem1):
        wid = lax.axis_index("s") * NC + lax.axis_index("c")
        w_base = elem_lo + wid * e_per_w
        idx_b, rows_b, sems = [idx0, idx1], [rows0, rows1], [sem0, sem1]
        srcs = [xw_hbm, xb_hbm]

        def fire(buf, chunk):
            @pl.when(chunk % IDXC == 0)
            def _():
                idx_base = (w_base + chunk * CHUNK_E) * L
                pltpu.sync_copy(
                    srcs[buf].at[pl.ds(idx_base, IDXC * ROWS)], idx_b[buf])

            slot = (chunk % IDXC) * ROWS
            off = 0
            while off < ROWS:
                n = min(128, ROWS - off)
                pltpu.async_copy(
                    table_hbm.at[idx_b[buf].at[pl.ds(slot + off, n)]],
                    rows_b[buf].at[pl.ds(off, n)], sems[buf])
                off += n

        def drain(buf):
            pltpu.make_async_copy(
                table_hbm.at[pl.ds(0, ROWS)], rows_b[buf], sems[buf]).wait()

        def unpack_row(rows_v, r):
            acc = []
            for g in range(D // 32):
                packed = plsc.bitcast(
                    rows_v[r, pl.ds(g * 16, 16)], jnp.bfloat16)
                lo, hi = plsc.unpack(
                    packed, format=plsc.PackFormat.INTERLEAVED)
                acc += [lo, hi]
            return tuple(acc)

        def reduce_side(buf):
            rows_v = rows_b[buf]
            for e in range(CHUNK_E):
                r0 = e * L

                def body7(t, acc, r0=r0, rows_v=rows_v):
                    j = 1 + t * 7
                    for u in range(7):
                        vals = unpack_row(rows_v, r0 + j + u)
                        acc = tuple(a + v for a, v in zip(acc, vals))
                    return acc

                acc = unpack_row(rows_v, r0)
                acc = lax.fori_loop(0, (L - 1) // 7, body7, acc)
                for d in range(D // 16):
                    outc_v[buf, e, pl.ds(d * 16, 16)] = acc[d]

        fire(0, 0)

        def chunk_body(c, carry):
            fire(1, c)
            drain(0)
            reduce_side(0)
            fire(0, jnp.minimum(c + 1, n_chunks - 1))
            drain(1)
            reduce_side(1)
            pltpu.sync_copy(outc_v, out_hbm.at[wid * n_chunks + c])
            return carry

        lax.fori_loop(0, n_chunks, chunk_body, 0)
        drain(0)

    return k(xw_flat, xb_flat, table_i32)


def _mlp_tc(x4, W2, b2, W3, b3, W4, b4):
    BLK = 2048
    BC = BLK // CHUNK_E

    def body(x_ref, w2_ref, b2_ref, w3_ref, b3_ref, w4_ref, b4_ref, o_ref):
        xw = jnp.maximum(x_ref[:, 0].reshape(BLK, D), 0.0)
        xb = jnp.maximum(x_ref[:, 1].reshape(BLK, D), 0.0)
        h = (jnp.dot(xw, w2_ref[:D], preferred_element_type=jnp.float32)
             + jnp.dot(xb, w2_ref[D:], preferred_element_type=jnp.float32))
        h = jnp.maximum(h + b2_ref[...], 0.0)
        h = jnp.dot(h, w3_ref[...], preferred_element_type=jnp.float32)
        h = jnp.maximum(h + b3_ref[...], 0.0)
        h = jnp.dot(h, w4_ref[...], preferred_element_type=jnp.float32)
        o_ref[...] = h + b4_ref[...]

    nb = x4.shape[0] * CHUNK_E
    return pl.pallas_call(
        body,
        grid=(nb // BLK,),
        in_specs=[
            pl.BlockSpec((BC, 2, CHUNK_E, D), lambda i: (i, 0, 0, 0)),
            pl.BlockSpec((2 * D, 32), lambda i: (0, 0)),
            pl.BlockSpec((1, 32), lambda i: (0, 0)),
            pl.BlockSpec((32, 32), lambda i: (0, 0)),
            pl.BlockSpec((1, 32), lambda i: (0, 0)),
            pl.BlockSpec((32, 1), lambda i: (0, 0)),
            pl.BlockSpec((1, 1), lambda i: (0, 0)),
        ],
        out_specs=pl.BlockSpec((BLK, 1), lambda i: (i, 0)),
        out_shape=jax.ShapeDtypeStruct((nb, 1), jnp.float32),
    )(x4, W2, b2.reshape(1, 32), W3, b3.reshape(1, 32), W4, b4.reshape(1, 1))


def kernel(x_w, x_b, table, W2, b2, W3, b3, W4, b4):
    xw_flat = x_w.astype(jnp.int32).reshape(-1)
    xb_flat = x_b.astype(jnp.int32).reshape(-1)
    table_i32 = _pack_table_sc(table.reshape(-1))
    halves = []
    for h in range(2):
        pooled = _emb_pool_sc(xw_flat, xb_flat, table_i32,
                              h * (B // 2), B // 2)
        halves.append(_mlp_tc(pooled, W2, b2, W3, b3, W4, b4))
    return jnp.concatenate(halves, axis=0)


# speedup vs baseline: 1.0264x; 1.0007x over previous
import functools

import jax
import jax.numpy as jnp
from jax import lax
from jax.experimental import pallas as pl
from jax.experimental.pallas import tpu as pltpu
from jax.experimental.pallas import tpu_sc as plsc

B = 16384
L = 50
D = 128
NC = 2
NS = 16
NW = NC * NS
E_PER_W = B // NW
CHUNK_E = 8
ROWS = CHUNK_E * L
N_CHUNKS = E_PER_W // CHUNK_E
IDXC = 4

VOCAB = 40961
PACK_CHUNK = 64
PACK_PER_W = -(-VOCAB // NW)
PACK_STEPS = -(-PACK_PER_W // PACK_CHUNK)


def _pack_table_sc(table_flat):
    mesh = plsc.VectorSubcoreMesh(core_axis_name="c", subcore_axis_name="s")

    @functools.partial(
        pl.kernel,
        out_type=jax.ShapeDtypeStruct((VOCAB, D // 2), jnp.int32),
        mesh=mesh,
        compiler_params=pltpu.CompilerParams(
            needs_layout_passes=False, use_tc_tiling_on_sc=False),
        scratch_types=[
            pltpu.VMEM((PACK_CHUNK * D,), jnp.float32),
            pltpu.VMEM((PACK_CHUNK * D,), jnp.float32),
            pltpu.VMEM((PACK_CHUNK, D // 2), jnp.int32),
            pltpu.VMEM((PACK_CHUNK, D // 2), jnp.int32),
            pltpu.SemaphoreType.DMA,
            pltpu.SemaphoreType.DMA,
            pltpu.SemaphoreType.DMA,
            pltpu.SemaphoreType.DMA,
        ],
    )
    def k(tflat_hbm, out_hbm, fin0, fin1, pout0, pout1,
          semA, semB, semOA, semOB):
        wid = lax.axis_index("s") * NC + lax.axis_index("c")
        fins, sems = [fin0, fin1], [semA, semB]
        pouts, osems = [pout0, pout1], [semOA, semOB]

        def src_row(t):
            return jnp.minimum(wid * PACK_PER_W + t * PACK_CHUNK,
                               VOCAB - PACK_CHUNK)

        def fire(t, buf):
            pltpu.async_copy(
                tflat_hbm.at[pl.ds(src_row(t) * D, PACK_CHUNK * D)],
                fins[buf], sems[buf])

        def drain(buf):
            pltpu.make_async_copy(
                tflat_hbm.at[pl.ds(0, PACK_CHUNK * D)],
                fins[buf], sems[buf]).wait()

        def drain_out(buf):
            pltpu.make_async_copy(
                pouts[buf], out_hbm.at[pl.ds(0, PACK_CHUNK)],
                osems[buf]).wait()

        def pack_step(fin_v, pout_v):
            half = jnp.full((16,), 0x8000, jnp.int32)
            lo_mask = jnp.full((16,), 0xFFFF, jnp.int32)
            hi_mask = jnp.full((16,), -65536, jnp.int32)
            for r in range(PACK_CHUNK):
                for g in range(D // 32):
                    a = plsc.bitcast(
                        fin_v[pl.ds(r * D + g * 32, 16)], jnp.int32)
                    b = plsc.bitcast(
                        fin_v[pl.ds(r * D + g * 32 + 16, 16)], jnp.int32)
                    a_bf = jnp.bitwise_and((a + half) >> 16, lo_mask)
                    b_bf = jnp.bitwise_and(b + half, hi_mask)
                    pout_v[r, pl.ds(g * 16, 16)] = jnp.bitwise_or(a_bf, b_bf)

        fire(0, 0)

        def step2(u, carry):
            for par in range(2):
                t = 2 * u + par
                fire(jnp.minimum(t + 1, PACK_STEPS - 1), 1 - par)
                drain(par)

                @pl.when(u > 0)
                def _():
                    drain_out(par)

                pack_step(fins[par], pouts[par])
                pltpu.async_copy(pouts[par],
                                 out_hbm.at[pl.ds(src_row(t), PACK_CHUNK)],
                                 osems[par])
            return carry

        lax.fori_loop(0, PACK_STEPS // 2, step2, 0)
        t = PACK_STEPS - 1
        drain(t % 2)
        drain_out(0)
        pack_step(fins[t % 2], pouts[0])
        pltpu.sync_copy(pouts[0], out_hbm.at[pl.ds(src_row(t), PACK_CHUNK)])
        drain_out(1)

    return k(table_flat)


def _emb_pool_sc(xw_flat, xb_flat, table_i32, elem_lo, nb):
    e_per_w = nb // NW
    n_chunks = e_per_w // CHUNK_E
    mesh = plsc.VectorSubcoreMesh(core_axis_name="c", subcore_axis_name="s")

    @functools.partial(
        pl.kernel,
        out_type=jax.ShapeDtypeStruct((nb // CHUNK_E, 2, CHUNK_E, D),
                                      jnp.float32),
        mesh=mesh,
        compiler_params=pltpu.CompilerParams(
            needs_layout_passes=False, use_tc_tiling_on_sc=False),
        scratch_types=[
            pltpu.VMEM((IDXC * ROWS,), jnp.int32),
            pltpu.VMEM((IDXC * ROWS,), jnp.int32),
            pltpu.VMEM((ROWS, D // 2), jnp.int32),
            pltpu.VMEM((ROWS, D // 2), jnp.int32),
            pltpu.VMEM((2, CHUNK_E, D), jnp.float32),
            pltpu.SemaphoreType.DMA,
            pltpu.SemaphoreType.DMA,
        ],
    )
    def k(xw_hbm, xb_hbm, table_hbm, out_hbm,
          idx0, idx1, rows0, rows1, outc_v, sem0, sem1):
        wid = lax.axis_index("s") * NC + lax.axis_index("c")
        w_base = elem_lo + wid * e_per_w
        idx_b, rows_b, sems = [idx0, idx1], [rows0, rows1], [sem0, sem1]
        srcs = [xw_hbm, xb_hbm]

        def fire(buf, chunk):
            @pl.when(chunk % IDXC == 0)
            def _():
                idx_base = (w_base + chunk * CHUNK_E) * L
                pltpu.sync_copy(
                    srcs[buf].at[pl.ds(idx_base, IDXC * ROWS)], idx_b[buf])

            slot = (chunk % IDXC) * ROWS
            off = 0
            while off < ROWS:
                n = min(128, ROWS - off)
                pltpu.async_copy(
                    table_hbm.at[idx_b[buf].at[pl.ds(slot + off, n)]],
                    rows_b[buf].at[pl.ds(off, n)], sems[buf])
                off += n

        def drain(buf):
            pltpu.make_async_copy(
                table_hbm.at[pl.ds(0, ROWS)], rows_b[buf], sems[buf]).wait()

        def unpack_row(rows_v, r):
            acc = []
            for g in range(D // 32):
                packed = plsc.bitcast(
                    rows_v[r, pl.ds(g * 16, 16)], jnp.bfloat16)
                lo, hi = plsc.unpack(
                    packed, format=plsc.PackFormat.INTERLEAVED)
                acc += [lo, hi]
            return tuple(acc)

        def reduce_side(buf):
            rows_v = rows_b[buf]
            for e in range(CHUNK_E):
                r0 = e * L

                def body7(t, acc, r0=r0, rows_v=rows_v):
                    j = 1 + t * 7
                    for u in range(7):
                        vals = unpack_row(rows_v, r0 + j + u)
                        acc = tuple(a + v for a, v in zip(acc, vals))
                    return acc

                acc = unpack_row(rows_v, r0)
                acc = lax.fori_loop(0, (L - 1) // 7, body7, acc)
                for d in range(D // 16):
                    outc_v[buf, e, pl.ds(d * 16, 16)] = acc[d]

        fire(0, 0)

        def chunk_body(c, carry):
            fire(1, c)
            drain(0)
            reduce_side(0)
            fire(0, jnp.minimum(c + 1, n_chunks - 1))
            drain(1)
            reduce_side(1)
            pltpu.sync_copy(outc_v, out_hbm.at[wid * n_chunks + c])
            return carry

        lax.fori_loop(0, n_chunks, chunk_body, 0)
        drain(0)

    return k(xw_flat, xb_flat, table_i32)


def _mlp_tc(x4, W2, b2, W3, b3, W4, b4):
    BLK = 2048
    BC = BLK // CHUNK_E

    def body(x_ref, w2_ref, b2_ref, w3_ref, b3_ref, w4_ref, b4_ref, o_ref):
        xw = jnp.maximum(x_ref[:, 0].reshape(BLK, D), 0.0)
        xb = jnp.maximum(x_ref[:, 1].reshape(BLK, D), 0.0)
        h = (jnp.dot(xw, w2_ref[:D], preferred_element_type=jnp.float32)
             + jnp.dot(xb, w2_ref[D:], preferred_element_type=jnp.float32))
        h = jnp.maximum(h + b2_ref[...], 0.0)
        h = jnp.dot(h, w3_ref[...], preferred_element_type=jnp.float32)
        h = jnp.maximum(h + b3_ref[...], 0.0)
        h = jnp.dot(h, w4_ref[...], preferred_element_type=jnp.float32)
        o_ref[...] = h + b4_ref[...]

    nb = x4.shape[0] * CHUNK_E
    return pl.pallas_call(
        body,
        grid=(nb // BLK,),
        in_specs=[
            pl.BlockSpec((BC, 2, CHUNK_E, D), lambda i: (i, 0, 0, 0)),
            pl.BlockSpec((2 * D, 32), lambda i: (0, 0)),
            pl.BlockSpec((1, 32), lambda i: (0, 0)),
            pl.BlockSpec((32, 32), lambda i: (0, 0)),
            pl.BlockSpec((1, 32), lambda i: (0, 0)),
            pl.BlockSpec((32, 1), lambda i: (0, 0)),
            pl.BlockSpec((1, 1), lambda i: (0, 0)),
        ],
        out_specs=pl.BlockSpec((BLK, 1), lambda i: (i, 0)),
        out_shape=jax.ShapeDtypeStruct((nb, 1), jnp.float32),
    )(x4, W2, b2.reshape(1, 32), W3, b3.reshape(1, 32), W4, b4.reshape(1, 1))


def kernel(x_w, x_b, table, W2, b2, W3, b3, W4, b4):
    xw_flat = x_w.astype(jnp.int32).reshape(-1)
    xb_flat = x_b.astype(jnp.int32).reshape(-1)
    table_i32 = _pack_table_sc(table.reshape(-1))
    halves = []
    for h in range(2):
        pooled = _emb_pool_sc(xw_flat, xb_flat, table_i32,
                              h * (B // 2), B // 2)
        halves.append(_mlp_tc(pooled, W2, b2, W3, b3, W4, b4))
    return jnp.concatenate(halves, axis=0)
